# baseline (speedup 1.0000x reference)
import functools

import jax
import jax.numpy as jnp
from jax import lax
from jax.experimental import pallas as pl
from jax.experimental.pallas import tpu as pltpu
from jax.experimental.pallas import tpu_sc as plsc


def _dot_t(a, b):
    return lax.dot_general(a, b, (((1,), (1,)), ((), ())),
                           preferred_element_type=jnp.float32)


def _pre_body(cb_ref, wu_ref, ct_ref, cup_ref, cn_ref, iota_ref):
    cb = cb_ref[...]
    cup_ref[...] = _dot_t(cb, wu_ref[...])
    ct = ct_ref[...]
    cn_ref[...] = jnp.sum(ct * ct, axis=0, keepdims=True)
    K = cb.shape[0]
    iota_ref[...] = lax.broadcasted_iota(
        jnp.int32, (1, K), 1).astype(jnp.float32)


def _precompute(codebook, W_up, ct):
    K, DC = codebook.shape
    DIN = W_up.shape[0]
    return pl.pallas_call(
        _pre_body,
        out_shape=[
            jax.ShapeDtypeStruct((K, DIN), jnp.float32),
            jax.ShapeDtypeStruct((1, K), jnp.float32),
            jax.ShapeDtypeStruct((1, K), jnp.float32),
        ],
    )(codebook, W_up, ct)


_BPS = 8


def _vq_body(z_ref, wd_ref, ct_ref, cn_ref, iota_ref,
             zd_ref, code_ref, loss_ref):
    T, DC = zd_ref.shape[1], zd_ref.shape[2]
    TB = T // _BPS
    K = ct_ref.shape[1]
    zd = _dot_t(z_ref[0], wd_ref[...])
    zd_ref[0] = zd
    scores = jnp.dot(zd, ct_ref[...],
                     preferred_element_type=jnp.float32)
    znorm = jnp.sum(zd * zd, axis=1, keepdims=True)
    dist = znorm - 2.0 * scores + cn_ref[...]
    m = jnp.min(dist, axis=1, keepdims=True)
    hit = jnp.where(dist <= m, iota_ref[...], jnp.float32(K))
    code_ref[0, 0] = jnp.min(hit, axis=1).astype(jnp.int32)
    inv = 1.0 / (TB * DC)
    for i in range(_BPS):
        li = jnp.sum(m[i * TB:(i + 1) * TB]) * inv
        loss_ref[0, i] = jnp.full((128,), li, jnp.float32)


def _vq_quantize(z_e, W_down, ct, cn, iota):
    B, T0, DIN = z_e.shape
    DC, K = ct.shape
    G = B // _BPS
    T = T0 * _BPS
    z_r = z_e.reshape(G, T, DIN)
    zd, code3, loss3 = pl.pallas_call(
        _vq_body,
        grid=(G,),
        in_specs=[
            pl.BlockSpec((1, T, DIN), lambda b: (b, 0, 0)),
            pl.BlockSpec((DC, DIN), lambda b: (0, 0)),
            pl.BlockSpec((DC, K), lambda b: (0, 0)),
            pl.BlockSpec((1, K), lambda b: (0, 0)),
            pl.BlockSpec((1, K), lambda b: (0, 0)),
        ],
        out_specs=[
            pl.BlockSpec((1, T, DC), lambda b: (b, 0, 0)),
            pl.BlockSpec((1, 1, T), lambda b: (b, 0, 0)),
            pl.BlockSpec((1, _BPS, 128), lambda b: (b, 0, 0)),
        ],
        out_shape=[
            jax.ShapeDtypeStruct((G, T, DC), jnp.float32),
            jax.ShapeDtypeStruct((G, 1, T), jnp.int32),
            jax.ShapeDtypeStruct((G, _BPS, 128), jnp.float32),
        ],
    )(z_r, W_down, ct, cn, iota)
    return (zd.reshape(B, T0, DC), code3.reshape(B, T0),
            loss3.reshape(B, 128)[:, 0])


def _sc_gather(cup, code_flat):
    info = plsc.get_sparse_core_info()
    NC, NS = info.num_cores, info.num_subcores
    NW = NC * NS
    n = code_flat.shape[0]
    D = cup.shape[1]
    bpw = n // NW
    CH = 96
    nch = bpw // CH
    mesh = plsc.VectorSubcoreMesh(core_axis_name="c", subcore_axis_name="s")

    @functools.partial(
        pl.kernel,
        mesh=mesh,
        out_type=jax.ShapeDtypeStruct((n, D), jnp.float32),
        scratch_types=[
            pltpu.VMEM((bpw,), jnp.int32),
            pltpu.VMEM((2, CH, D), jnp.float32),
            pltpu.SemaphoreType.DMA,
            pltpu.SemaphoreType.DMA,
        ],
    )
    def k(cup_hbm, idx_hbm, out_hbm, idx_v, rows_v, gsem, ssem):
        wid = lax.axis_index("s") * NC + lax.axis_index("c")
        base = wid * bpw
        pltpu.sync_copy(idx_hbm.at[pl.ds(base, bpw)], idx_v)
        gathers = [None] * nch
        scatters = [None] * nch
        gathers[0] = pltpu.async_copy(
            cup_hbm.at[idx_v.at[pl.ds(0, CH)]], rows_v.at[0], gsem)
        for j in range(nch):
            gathers[j].wait()
            if j + 1 < nch:
                if j - 1 >= 0:
                    scatters[j - 1].wait()
                gathers[j + 1] = pltpu.async_copy(
                    cup_hbm.at[idx_v.at[pl.ds((j + 1) * CH, CH)]],
                    rows_v.at[(j + 1) % 2], gsem)
            scatters[j] = pltpu.async_copy(
                rows_v.at[j % 2], out_hbm.at[pl.ds(base + j * CH, CH)], ssem)
        scatters[nch - 2].wait()
        scatters[nch - 1].wait()

    return k(cup, code_flat)


def kernel(z_e, W_down, W_up, codebook):
    B, T, DIN = z_e.shape
    ct = codebook.T
    cup, cn, iota = _precompute(codebook, W_up, ct)
    zd, code, loss = _vq_quantize(z_e, W_down, ct, cn, iota)
    zq_flat = _sc_gather(cup, code.reshape(B * T))
    z_q = zq_flat.reshape(B, T, DIN)
    return (z_q, zd, code, loss, loss)


# speedup vs baseline: 1.0302x; 1.0302x over previous
import functools

import jax
import jax.numpy as jnp
from jax import lax
from jax.experimental import pallas as pl
from jax.experimental.pallas import tpu as pltpu
from jax.experimental.pallas import tpu_sc as plsc


def _dot_t(a, b):
    return lax.dot_general(a, b, (((1,), (1,)), ((), ())),
                           preferred_element_type=jnp.float32)


def _pre_body(cb_ref, wu_ref, cup_ref, ct_ref, cn_ref, iota_ref):
    cb = cb_ref[...]
    cup_ref[...] = _dot_t(cb, wu_ref[...])
    ct = jnp.transpose(cb)
    ct_ref[...] = ct
    cn_ref[...] = jnp.sum(ct * ct, axis=0, keepdims=True)
    K = cb.shape[0]
    iota_ref[...] = lax.broadcasted_iota(
        jnp.int32, (1, K), 1).astype(jnp.float32)


def _precompute(codebook, W_up):
    K, DC = codebook.shape
    DIN = W_up.shape[0]
    return pl.pallas_call(
        _pre_body,
        out_shape=[
            jax.ShapeDtypeStruct((K, DIN), jnp.float32),
            jax.ShapeDtypeStruct((DC, K), jnp.float32),
            jax.ShapeDtypeStruct((1, K), jnp.float32),
            jax.ShapeDtypeStruct((1, K), jnp.float32),
        ],
    )(codebook, W_up)


_BPS = 4


def _vq_body(z_ref, wd_ref, ct_ref, cn_ref, iota_ref,
             zd_ref, code_ref, loss_ref):
    T, DC = zd_ref.shape[1], zd_ref.shape[2]
    TB = T // _BPS
    K = ct_ref.shape[1]
    zd = _dot_t(z_ref[0], wd_ref[...])
    zd_ref[0] = zd
    scores = jnp.dot(zd, ct_ref[...],
                     preferred_element_type=jnp.float32)
    znorm = jnp.sum(zd * zd, axis=1, keepdims=True)
    dist = znorm - 2.0 * scores + cn_ref[...]
    m = jnp.min(dist, axis=1, keepdims=True)
    hit = jnp.where(dist <= m, iota_ref[...], jnp.float32(K))
    code_ref[0, 0] = jnp.min(hit, axis=1).astype(jnp.int32)
    inv = 1.0 / (TB * DC)
    for i in range(_BPS):
        li = jnp.sum(m[i * TB:(i + 1) * TB]) * inv
        loss_ref[0, i] = jnp.full((128,), li, jnp.float32)


def _vq_quantize(z_e, W_down, ct, cn, iota):
    B, T0, DIN = z_e.shape
    DC, K = ct.shape
    G = B // _BPS
    T = T0 * _BPS
    z_r = z_e.reshape(G, T, DIN)
    zd, code3, loss3 = pl.pallas_call(
        _vq_body,
        grid=(G,),
        in_specs=[
            pl.BlockSpec((1, T, DIN), lambda b: (b, 0, 0)),
            pl.BlockSpec((DC, DIN), lambda b: (0, 0)),
            pl.BlockSpec((DC, K), lambda b: (0, 0)),
            pl.BlockSpec((1, K), lambda b: (0, 0)),
            pl.BlockSpec((1, K), lambda b: (0, 0)),
        ],
        out_specs=[
            pl.BlockSpec((1, T, DC), lambda b: (b, 0, 0)),
            pl.BlockSpec((1, 1, T), lambda b: (b, 0, 0)),
            pl.BlockSpec((1, _BPS, 128), lambda b: (b, 0, 0)),
        ],
        out_shape=[
            jax.ShapeDtypeStruct((G, T, DC), jnp.float32),
            jax.ShapeDtypeStruct((G, 1, T), jnp.int32),
            jax.ShapeDtypeStruct((G, _BPS, 128), jnp.float32),
        ],
    )(z_r, W_down, ct, cn, iota)
    return (zd.reshape(B, T0, DC), code3.reshape(B, T0),
            loss3.reshape(B, 128)[:, 0])


def _sc_gather(cup, code_flat):
    info = plsc.get_sparse_core_info()
    NC, NS = info.num_cores, info.num_subcores
    NW = NC * NS
    n = code_flat.shape[0]
    D = cup.shape[1]
    bpw = n // NW
    CH = 96
    nch = bpw // CH
    mesh = plsc.VectorSubcoreMesh(core_axis_name="c", subcore_axis_name="s")

    @functools.partial(
        pl.kernel,
        mesh=mesh,
        out_type=jax.ShapeDtypeStruct((n, D), jnp.float32),
        scratch_types=[
            pltpu.VMEM((bpw,), jnp.int32),
            pltpu.VMEM((2, CH, D), jnp.float32),
            pltpu.SemaphoreType.DMA,
            pltpu.SemaphoreType.DMA,
        ],
    )
    def k(cup_hbm, idx_hbm, out_hbm, idx_v, rows_v, gsem, ssem):
        wid = lax.axis_index("s") * NC + lax.axis_index("c")
        base = wid * bpw
        pltpu.sync_copy(idx_hbm.at[pl.ds(base, bpw)], idx_v)
        gathers = [None] * nch
        scatters = [None] * nch
        gathers[0] = pltpu.async_copy(
            cup_hbm.at[idx_v.at[pl.ds(0, CH)]], rows_v.at[0], gsem)
        for j in range(nch):
            gathers[j].wait()
            if j + 1 < nch:
                if j - 1 >= 0:
                    scatters[j - 1].wait()
                gathers[j + 1] = pltpu.async_copy(
                    cup_hbm.at[idx_v.at[pl.ds((j + 1) * CH, CH)]],
                    rows_v.at[(j + 1) % 2], gsem)
            scatters[j] = pltpu.async_copy(
                rows_v.at[j % 2], out_hbm.at[pl.ds(base + j * CH, CH)], ssem)
        scatters[nch - 2].wait()
        scatters[nch - 1].wait()

    return k(cup, code_flat)


def kernel(z_e, W_down, W_up, codebook):
    B, T, DIN = z_e.shape
    cup, ct, cn, iota = _precompute(codebook, W_up)
    zd, code, loss = _vq_quantize(z_e, W_down, ct, cn, iota)
    zq_flat = _sc_gather(cup, code.reshape(B * T))
    z_q = zq_flat.reshape(B, T, DIN)
    return (z_q, zd, code, loss, loss)


# speedup vs baseline: 1.0316x; 1.0014x over previous
import functools

import jax
import jax.numpy as jnp
from jax import lax
from jax.experimental import pallas as pl
from jax.experimental.pallas import tpu as pltpu
from jax.experimental.pallas import tpu_sc as plsc


def _dot_t(a, b):
    return lax.dot_general(a, b, (((1,), (1,)), ((), ())),
                           preferred_element_type=jnp.float32)


def _pre_body(cb_ref, wu_ref, cup_ref, ct_ref, cn_ref, iota_ref):
    cb = cb_ref[...]
    cup_ref[...] = _dot_t(cb, wu_ref[...])
    ct = jnp.transpose(cb)
    ct_ref[...] = ct
    cn_ref[...] = jnp.sum(ct * ct, axis=0, keepdims=True)
    K = cb.shape[0]
    iota_ref[...] = lax.broadcasted_iota(
        jnp.int32, (1, K), 1).astype(jnp.float32)


def _precompute(codebook, W_up):
    K, DC = codebook.shape
    DIN = W_up.shape[0]
    return pl.pallas_call(
        _pre_body,
        out_shape=[
            jax.ShapeDtypeStruct((K, DIN), jnp.float32),
            jax.ShapeDtypeStruct((DC, K), jnp.float32),
            jax.ShapeDtypeStruct((1, K), jnp.float32),
            jax.ShapeDtypeStruct((1, K), jnp.float32),
        ],
    )(codebook, W_up)


_BPS = 4


def _vq_body(z_ref, wd_ref, ct_ref, cn_ref, iota_ref,
             zd_ref, code_ref, loss_ref):
    T, DC = zd_ref.shape[1], zd_ref.shape[2]
    TB = T // _BPS
    K = ct_ref.shape[1]
    zd = _dot_t(z_ref[0], wd_ref[...])
    zd_ref[0] = zd
    scores = jnp.dot(zd, ct_ref[...],
                     preferred_element_type=jnp.float32)
    znorm = jnp.sum(zd * zd, axis=1, keepdims=True)
    dist = znorm - 2.0 * scores + cn_ref[...]
    m = jnp.min(dist, axis=1, keepdims=True)
    hit = jnp.where(dist <= m, iota_ref[...], jnp.float32(K))
    code_ref[0, 0] = jnp.min(hit, axis=1).astype(jnp.int32)
    inv = 1.0 / (TB * DC)
    sub = lax.broadcasted_iota(jnp.int32, (_BPS, T), 0)
    tok = lax.broadcasted_iota(jnp.int32, (_BPS, T), 1)
    lo = sub * TB
    sel = ((tok >= lo) & (tok < lo + TB)).astype(jnp.float32)
    lsum = jnp.dot(sel, m, preferred_element_type=jnp.float32)
    loss_ref[0] = jnp.broadcast_to(lsum * inv, (_BPS, 128))


def _vq_quantize(z_e, W_down, ct, cn, iota):
    B, T0, DIN = z_e.shape
    DC, K = ct.shape
    G = B // _BPS
    T = T0 * _BPS
    z_r = z_e.reshape(G, T, DIN)
    zd, code3, loss3 = pl.pallas_call(
        _vq_body,
        grid=(G,),
        in_specs=[
            pl.BlockSpec((1, T, DIN), lambda b: (b, 0, 0)),
            pl.BlockSpec((DC, DIN), lambda b: (0, 0)),
            pl.BlockSpec((DC, K), lambda b: (0, 0)),
            pl.BlockSpec((1, K), lambda b: (0, 0)),
            pl.BlockSpec((1, K), lambda b: (0, 0)),
        ],
        out_specs=[
            pl.BlockSpec((1, T, DC), lambda b: (b, 0, 0)),
            pl.BlockSpec((1, 1, T), lambda b: (b, 0, 0)),
            pl.BlockSpec((1, _BPS, 128), lambda b: (b, 0, 0)),
        ],
        out_shape=[
            jax.ShapeDtypeStruct((G, T, DC), jnp.float32),
            jax.ShapeDtypeStruct((G, 1, T), jnp.int32),
            jax.ShapeDtypeStruct((G, _BPS, 128), jnp.float32),
        ],
    )(z_r, W_down, ct, cn, iota)
    return (zd.reshape(B, T0, DC), code3.reshape(B, T0),
            loss3.reshape(B, 128)[:, 0])


def _sc_gather(cup, code_flat):
    info = plsc.get_sparse_core_info()
    NC, NS = info.num_cores, info.num_subcores
    NW = NC * NS
    n = code_flat.shape[0]
    D = cup.shape[1]
    bpw = n // NW
    CH = 96
    nch = bpw // CH
    mesh = plsc.VectorSubcoreMesh(core_axis_name="c", subcore_axis_name="s")

    @functools.partial(
        pl.kernel,
        mesh=mesh,
        out_type=jax.ShapeDtypeStruct((n, D), jnp.float32),
        scratch_types=[
            pltpu.VMEM((bpw,), jnp.int32),
            pltpu.VMEM((2, CH, D), jnp.float32),
            pltpu.SemaphoreType.DMA,
            pltpu.SemaphoreType.DMA,
        ],
    )
    def k(cup_hbm, idx_hbm, out_hbm, idx_v, rows_v, gsem, ssem):
        wid = lax.axis_index("s") * NC + lax.axis_index("c")
        base = wid * bpw
        pltpu.sync_copy(idx_hbm.at[pl.ds(base, bpw)], idx_v)
        gathers = [None] * nch
        scatters = [None] * nch
        gathers[0] = pltpu.async_copy(
            cup_hbm.at[idx_v.at[pl.ds(0, CH)]], rows_v.at[0], gsem)
        for j in range(nch):
            gathers[j].wait()
            if j + 1 < nch:
                if j - 1 >= 0:
                    scatters[j - 1].wait()
                gathers[j + 1] = pltpu.async_copy(
                    cup_hbm.at[idx_v.at[pl.ds((j + 1) * CH, CH)]],
                    rows_v.at[(j + 1) % 2], gsem)
            scatters[j] = pltpu.async_copy(
                rows_v.at[j % 2], out_hbm.at[pl.ds(base + j * CH, CH)], ssem)
        scatters[nch - 2].wait()
        scatters[nch - 1].wait()

    return k(cup, code_flat)


def kernel(z_e, W_down, W_up, codebook):
    B, T, DIN = z_e.shape
    cup, ct, cn, iota = _precompute(codebook, W_up)
    zd, code, loss = _vq_quantize(z_e, W_down, ct, cn, iota)
    zq_flat = _sc_gather(cup, code.reshape(B * T))
    z_q = zq_flat.reshape(B, T, DIN)
    return (z_q, zd, code, loss, loss)


# speedup vs baseline: 1.1112x; 1.0772x over previous
import functools

import jax
import jax.numpy as jnp
from jax import lax
from jax.experimental import pallas as pl
from jax.experimental.pallas import tpu as pltpu
from jax.experimental.pallas import tpu_sc as plsc


def _dot_t(a, b):
    return lax.dot_general(a, b, (((1,), (1,)), ((), ())),
                           preferred_element_type=jnp.float32)


def _pre_body(cb_ref, wu_ref, cup_ref, ct_ref, cn_ref, iota_ref):
    cb = cb_ref[...]
    cup_ref[...] = _dot_t(cb, wu_ref[...])
    ct = jnp.transpose(cb)
    ct_ref[...] = ct
    cn_ref[...] = jnp.sum(ct * ct, axis=0, keepdims=True)
    K = cb.shape[0]
    iota_ref[...] = lax.broadcasted_iota(
        jnp.int32, (1, K), 1).astype(jnp.float32)


def _precompute(codebook, W_up):
    K, DC = codebook.shape
    DIN = W_up.shape[0]
    return pl.pallas_call(
        _pre_body,
        out_shape=[
            jax.ShapeDtypeStruct((K, DIN), jnp.float32),
            jax.ShapeDtypeStruct((DC, K), jnp.float32),
            jax.ShapeDtypeStruct((1, K), jnp.float32),
            jax.ShapeDtypeStruct((1, K), jnp.float32),
        ],
    )(codebook, W_up)


_BPS = 4


def _vq_body(z_ref, wd_ref, ct_ref, cn_ref, iota_ref,
             zd_ref, code_ref, loss_ref):
    T, DC = zd_ref.shape[1], zd_ref.shape[2]
    TB = T // _BPS
    K = ct_ref.shape[1]
    zd = _dot_t(z_ref[0], wd_ref[...])
    zd_ref[0] = zd
    H = T // 4
    ms = []
    for h in range(4):
        zh = zd[h * H:(h + 1) * H]
        s = jnp.dot(zh, ct_ref[...],
                    preferred_element_type=jnp.float32)
        zn = jnp.sum(zh * zh, axis=1, keepdims=True)
        dist = zn - 2.0 * s + cn_ref[...]
        m = jnp.min(dist, axis=1, keepdims=True)
        hit = jnp.where(dist <= m, iota_ref[...], jnp.float32(K))
        code_ref[0, 0, h * H:(h + 1) * H] = jnp.min(hit, axis=1).astype(
            jnp.int32)
        ms.append(m)
    m = jnp.concatenate(ms, axis=0)
    inv = 1.0 / (TB * DC)
    sub = lax.broadcasted_iota(jnp.int32, (_BPS, T), 0)
    tok = lax.broadcasted_iota(jnp.int32, (_BPS, T), 1)
    lo = sub * TB
    sel = ((tok >= lo) & (tok < lo + TB)).astype(jnp.float32)
    lsum = jnp.dot(sel, m, preferred_element_type=jnp.float32)
    loss_ref[0] = jnp.broadcast_to(lsum * inv, (_BPS, 128))


def _vq_quantize(z_e, W_down, ct, cn, iota):
    B, T0, DIN = z_e.shape
    DC, K = ct.shape
    G = B // _BPS
    T = T0 * _BPS
    z_r = z_e.reshape(G, T, DIN)
    zd, code3, loss3 = pl.pallas_call(
        _vq_body,
        grid=(G,),
        in_specs=[
            pl.BlockSpec((1, T, DIN), lambda b: (b, 0, 0)),
            pl.BlockSpec((DC, DIN), lambda b: (0, 0)),
            pl.BlockSpec((DC, K), lambda b: (0, 0)),
            pl.BlockSpec((1, K), lambda b: (0, 0)),
            pl.BlockSpec((1, K), lambda b: (0, 0)),
        ],
        out_specs=[
            pl.BlockSpec((1, T, DC), lambda b: (b, 0, 0)),
            pl.BlockSpec((1, 1, T), lambda b: (b, 0, 0)),
            pl.BlockSpec((1, _BPS, 128), lambda b: (b, 0, 0)),
        ],
        out_shape=[
            jax.ShapeDtypeStruct((G, T, DC), jnp.float32),
            jax.ShapeDtypeStruct((G, 1, T), jnp.int32),
            jax.ShapeDtypeStruct((G, _BPS, 128), jnp.float32),
        ],
    )(z_r, W_down, ct, cn, iota)
    return (zd.reshape(B, T0, DC), code3.reshape(B, T0),
            loss3.reshape(B, 128)[:, 0])


def _sc_gather(cup, code_flat):
    info = plsc.get_sparse_core_info()
    NC, NS = info.num_cores, info.num_subcores
    NW = NC * NS
    n = code_flat.shape[0]
    D = cup.shape[1]
    bpw = n // NW
    CH = 96
    nch = bpw // CH
    mesh = plsc.VectorSubcoreMesh(core_axis_name="c", subcore_axis_name="s")

    @functools.partial(
        pl.kernel,
        mesh=mesh,
        out_type=jax.ShapeDtypeStruct((n, D), jnp.float32),
        scratch_types=[
            pltpu.VMEM((bpw,), jnp.int32),
            pltpu.VMEM((2, CH, D), jnp.float32),
            pltpu.SemaphoreType.DMA,
            pltpu.SemaphoreType.DMA,
        ],
    )
    def k(cup_hbm, idx_hbm, out_hbm, idx_v, rows_v, gsem, ssem):
        wid = lax.axis_index("s") * NC + lax.axis_index("c")
        base = wid * bpw
        pltpu.sync_copy(idx_hbm.at[pl.ds(base, bpw)], idx_v)
        gathers = [None] * nch
        scatters = [None] * nch
        gathers[0] = pltpu.async_copy(
            cup_hbm.at[idx_v.at[pl.ds(0, CH)]], rows_v.at[0], gsem)
        for j in range(nch):
            gathers[j].wait()
            if j + 1 < nch:
                if j - 1 >= 0:
                    scatters[j - 1].wait()
                gathers[j + 1] = pltpu.async_copy(
                    cup_hbm.at[idx_v.at[pl.ds((j + 1) * CH, CH)]],
                    rows_v.at[(j + 1) % 2], gsem)
            scatters[j] = pltpu.async_copy(
                rows_v.at[j % 2], out_hbm.at[pl.ds(base + j * CH, CH)], ssem)
        scatters[nch - 2].wait()
        scatters[nch - 1].wait()

    return k(cup, code_flat)


def kernel(z_e, W_down, W_up, codebook):
    B, T, DIN = z_e.shape
    cup, ct, cn, iota = _precompute(codebook, W_up)
    zd, code, loss = _vq_quantize(z_e, W_down, ct, cn, iota)
    zq_flat = _sc_gather(cup, code.reshape(B * T))
    z_q = zq_flat.reshape(B, T, DIN)
    return (z_q, zd, code, loss, loss)
